# Initial kernel scaffold; baseline (speedup 1.0000x reference)
import functools

import jax
import jax.numpy as jnp
from jax import lax
from jax.experimental import pallas as pl
from jax.experimental.pallas import tpu as pltpu
from jax.experimental.pallas import tpu_sc as plsc

N = 10000
E = 320000
D = 128
DH = D // 2
NS = 16
EPT = E // NS
CH = 80
NCHUNK = EPT // CH
ROWS_PT = N // NS

_BLK = 2000


def _mm_body(h_ref, wl_ref, wr_ref, bl_ref, y_ref, z_ref):
    h = h_ref[...]
    dn = (((1,), (1,)), ((), ()))
    y_ref[...] = lax.dot_general(h, wl_ref[...], dn,
                                 preferred_element_type=jnp.float32)
    z_ref[...] = lax.dot_general(h, wr_ref[...], dn,
                                 preferred_element_type=jnp.float32) + bl_ref[...]


def _mm(h, wl, wr, bl):
    grid = (N // _BLK,)
    return pl.pallas_call(
        _mm_body,
        grid=grid,
        in_specs=[
            pl.BlockSpec((_BLK, D), lambda i: (i, 0)),
            pl.BlockSpec((D, D), lambda i: (0, 0)),
            pl.BlockSpec((D, D), lambda i: (0, 0)),
            pl.BlockSpec((1, D), lambda i: (0, 0)),
        ],
        out_specs=[
            pl.BlockSpec((_BLK, D), lambda i: (i, 0)),
            pl.BlockSpec((_BLK, D), lambda i: (i, 0)),
        ],
        out_shape=[
            jax.ShapeDtypeStruct((N, D), jnp.float32),
            jax.ShapeDtypeStruct((N, D), jnp.float32),
        ],
    )(h, wl, wr, bl.reshape(1, D))


def _combine_body(act, aggA_ref, aggB_ref, cnt_ref, z_ref, o_ref):
    cnt = jnp.maximum(cnt_ref[...], 1.0)
    agg = jnp.concatenate([aggA_ref[...], aggB_ref[...]], axis=1)
    h = agg / cnt + z_ref[...]
    if act == "relu":
        h = jnp.maximum(h, 0.0)
    elif act == "logsoftmax":
        m = jnp.max(h, axis=1, keepdims=True)
        h = h - m
        h = h - jnp.log(jnp.sum(jnp.exp(h), axis=1, keepdims=True))
    o_ref[...] = h


def _combine(agg2, cnt, z, act):
    grid = (N // _BLK,)
    nb = N // _BLK
    return pl.pallas_call(
        functools.partial(_combine_body, act),
        grid=grid,
        in_specs=[
            pl.BlockSpec((_BLK, DH), lambda i: (i, 0)),
            pl.BlockSpec((_BLK, DH), lambda i, nb=nb: (i + nb, 0)),
            pl.BlockSpec((_BLK, 1), lambda i: (i, 0)),
            pl.BlockSpec((_BLK, D), lambda i: (i, 0)),
        ],
        out_specs=pl.BlockSpec((_BLK, D), lambda i: (i, 0)),
        out_shape=jax.ShapeDtypeStruct((N, D), jnp.float32),
    )(agg2, agg2, cnt, z)


def _sc_body(ys_h, src_h, dst_h, zrows_h, zcnt_h,
             agg_h, cnt_h,
             acc_s, cntacc_s,
             src_v, dst_v, rows_v, ones_v):
    cid = lax.axis_index("c")
    sid = lax.axis_index("s")

    pltpu.sync_copy(zrows_h, acc_s.at[pl.ds(sid * ROWS_PT, ROWS_PT)])

    @pl.when(jnp.logical_and(cid == 0, sid == 0))
    def _():
        pltpu.sync_copy(zcnt_h, cntacc_s)

    pltpu.sync_copy(src_h.at[pl.ds(sid * NCHUNK, NCHUNK)], src_v)
    pltpu.sync_copy(dst_h.at[pl.ds(sid * NCHUNK, NCHUNK)], dst_v)

    for k in range(CH // 16):
        ones_v[pl.ds(k * 16, 16)] = jnp.full((16,), 1.0, jnp.float32)

    def remap(j, _):
        for k in range(CH // 16):
            sl = pl.ds(k * 16, 16)
            src_v[j, sl] = src_v[j, sl] * 2 + cid
        return 0

    lax.fori_loop(0, NCHUNK, remap, 0)

    plsc.subcore_barrier()

    def chunk(j, _):
        pltpu.sync_copy(ys_h.at[src_v.at[j]], rows_v)
        pltpu.sync_copy(rows_v, acc_s.at[dst_v.at[j]], add=True)

        @pl.when(cid == 0)
        def _():
            pltpu.sync_copy(ones_v, cntacc_s.at[dst_v.at[j]], add=True)

        return 0

    lax.fori_loop(0, NCHUNK, chunk, 0)

    plsc.subcore_barrier()

    pltpu.sync_copy(acc_s.at[pl.ds(sid * ROWS_PT, ROWS_PT)],
                    agg_h.at[pl.ds(cid * N + sid * ROWS_PT, ROWS_PT)])

    @pl.when(jnp.logical_and(cid == 0, sid < 10))
    def _():
        pltpu.sync_copy(cntacc_s.at[pl.ds(sid * 1000, 1000)],
                        cnt_h.at[pl.ds(sid * 1000, 1000)])


def _sc_agg(ys, src2, dst2, zrows, zcnt):
    mesh = plsc.VectorSubcoreMesh(core_axis_name="c", subcore_axis_name="s")
    f = pl.kernel(
        _sc_body,
        out_type=[
            jax.ShapeDtypeStruct((2 * N, DH), jnp.float32),
            jax.ShapeDtypeStruct((N,), jnp.float32),
        ],
        mesh=mesh,
        scratch_types=[
            pltpu.VMEM_SHARED((N, DH), jnp.float32),
            pltpu.VMEM_SHARED((N,), jnp.float32),
            pltpu.VMEM((NCHUNK, CH), jnp.int32),
            pltpu.VMEM((NCHUNK, CH), jnp.int32),
            pltpu.VMEM((CH, DH), jnp.float32),
            pltpu.VMEM((CH,), jnp.float32),
        ],
    )
    return f(ys, src2, dst2, zrows, zcnt)


def _layer(h, wl, wr, bl, act, src2, dst2, zrows, zcnt):
    y, z = _mm(h, wl, wr, bl)
    ys = y.reshape(2 * N, DH)
    agg2, cnt = _sc_agg(ys, src2, dst2, zrows, zcnt)
    return _combine(agg2, cnt.reshape(N, 1), z, act)


def kernel(x, edge_index, W1l, W1r, W2l, W2r, W3l, W3r, b1l, b2l, b3l):
    src2 = edge_index[0].reshape(E // CH, CH)
    dst2 = edge_index[1].reshape(E // CH, CH)
    zrows = jnp.zeros((ROWS_PT, DH), jnp.float32)
    zcnt = jnp.zeros((N,), jnp.float32)

    h = _layer(x, W1l, W1r, b1l, "relu", src2, dst2, zrows, zcnt)
    h = _layer(h, W2l, W2r, b2l, "relu", src2, dst2, zrows, zcnt)
    return _layer(h, W3l, W3r, b3l, "logsoftmax", src2, dst2, zrows, zcnt)


# speedup vs baseline: 6.9445x; 6.9445x over previous
import functools

import jax
import jax.numpy as jnp
from jax import lax
from jax.experimental import pallas as pl
from jax.experimental.pallas import tpu as pltpu
from jax.experimental.pallas import tpu_sc as plsc

N = 10000
E = 320000
D = 128
NS = 16
NW = 2 * NS
EPT = E // NW
CH = 80
NCHUNK = EPT // CH
NP = 10240
ROWS_PT = NP // NS

_BLK = 2000


def _mm_body(h_ref, wl_ref, wr_ref, bl_ref, y_ref, z_ref):
    h = h_ref[...]
    dn = (((1,), (1,)), ((), ()))
    y_ref[...] = lax.dot_general(h, wl_ref[...], dn,
                                 preferred_element_type=jnp.float32)
    z_ref[...] = lax.dot_general(h, wr_ref[...], dn,
                                 preferred_element_type=jnp.float32) + bl_ref[...]


def _mm(h, wl, wr, bl):
    grid = (N // _BLK,)
    return pl.pallas_call(
        _mm_body,
        grid=grid,
        in_specs=[
            pl.BlockSpec((_BLK, D), lambda i: (i, 0)),
            pl.BlockSpec((D, D), lambda i: (0, 0)),
            pl.BlockSpec((D, D), lambda i: (0, 0)),
            pl.BlockSpec((1, D), lambda i: (0, 0)),
        ],
        out_specs=[
            pl.BlockSpec((_BLK, D), lambda i: (i, 0)),
            pl.BlockSpec((_BLK, D), lambda i: (i, 0)),
        ],
        out_shape=[
            jax.ShapeDtypeStruct((N, D), jnp.float32),
            jax.ShapeDtypeStruct((N, D), jnp.float32),
        ],
    )(h, wl, wr, bl.reshape(1, D))


def _combine_body(act, aggA_ref, aggB_ref, cntA_ref, cntB_ref, z_ref, o_ref):
    cnt = jnp.maximum(cntA_ref[...] + cntB_ref[...], 1.0)
    agg = aggA_ref[0] + aggB_ref[0]
    h = agg / cnt + z_ref[...]
    if act == "relu":
        h = jnp.maximum(h, 0.0)
    elif act == "logsoftmax":
        m = jnp.max(h, axis=1, keepdims=True)
        h = h - m
        h = h - jnp.log(jnp.sum(jnp.exp(h), axis=1, keepdims=True))
    o_ref[...] = h


def _combine(agg2, cntA, cntB, z, act):
    grid = (N // _BLK,)
    return pl.pallas_call(
        functools.partial(_combine_body, act),
        grid=grid,
        in_specs=[
            pl.BlockSpec((1, _BLK, D), lambda i: (0, i, 0)),
            pl.BlockSpec((1, _BLK, D), lambda i: (1, i, 0)),
            pl.BlockSpec((_BLK, 1), lambda i: (i, 0)),
            pl.BlockSpec((_BLK, 1), lambda i: (i, 0)),
            pl.BlockSpec((_BLK, D), lambda i: (i, 0)),
        ],
        out_specs=pl.BlockSpec((_BLK, D), lambda i: (i, 0)),
        out_shape=jax.ShapeDtypeStruct((N, D), jnp.float32),
    )(agg2, agg2, cntA, cntB, z)


def _sc_body(ys_h, src_h, dst_h, zrows_h, zcnt_h,
             agg_h, cntA_h, cntB_h,
             acc_s, cntacc_s,
             src_v, dst_v, rows_v, ones_v):
    cid = lax.axis_index("c")
    sid = lax.axis_index("s")

    wid = cid * NS + sid

    pltpu.sync_copy(zrows_h, acc_s.at[pl.ds(sid * ROWS_PT, ROWS_PT)])
    pltpu.sync_copy(zcnt_h, cntacc_s.at[pl.ds(sid * ROWS_PT, ROWS_PT)])

    pltpu.sync_copy(src_h.at[wid], src_v)
    pltpu.sync_copy(dst_h.at[wid], dst_v)

    for k in range(CH // 16):
        ones_v[pl.ds(k * 16, 16)] = jnp.full((16,), 1.0, jnp.float32)

    plsc.subcore_barrier()

    def chunk(j, _):
        pltpu.sync_copy(ys_h.at[src_v.at[j]], rows_v)
        pltpu.sync_copy(rows_v, acc_s.at[dst_v.at[j]], add=True)
        pltpu.sync_copy(ones_v, cntacc_s.at[dst_v.at[j]], add=True)
        return 0

    lax.fori_loop(0, NCHUNK, chunk, 0)

    plsc.subcore_barrier()

    pltpu.sync_copy(acc_s.at[pl.ds(sid * ROWS_PT, ROWS_PT)],
                    agg_h.at[cid].at[pl.ds(sid * ROWS_PT, ROWS_PT)])

    @pl.when(cid == 0)
    def _():
        pltpu.sync_copy(cntacc_s.at[pl.ds(sid * ROWS_PT, ROWS_PT)],
                        cntA_h.at[pl.ds(sid * ROWS_PT, ROWS_PT)])

    @pl.when(cid == 1)
    def _():
        pltpu.sync_copy(cntacc_s.at[pl.ds(sid * ROWS_PT, ROWS_PT)],
                        cntB_h.at[pl.ds(sid * ROWS_PT, ROWS_PT)])


def _sc_agg(ys, src2, dst2, zrows, zcnt):
    mesh = plsc.VectorSubcoreMesh(core_axis_name="c", subcore_axis_name="s")
    f = pl.kernel(
        _sc_body,
        out_type=[
            jax.ShapeDtypeStruct((2, NP, D), jnp.float32),
            jax.ShapeDtypeStruct((NP,), jnp.float32),
            jax.ShapeDtypeStruct((NP,), jnp.float32),
        ],
        mesh=mesh,
        scratch_types=[
            pltpu.VMEM_SHARED((NP, D), jnp.float32),
            pltpu.VMEM_SHARED((NP,), jnp.float32),
            pltpu.VMEM((NCHUNK, CH), jnp.int32),
            pltpu.VMEM((NCHUNK, CH), jnp.int32),
            pltpu.VMEM((CH, D), jnp.float32),
            pltpu.VMEM((CH,), jnp.float32),
        ],
    )
    return f(ys, src2, dst2, zrows, zcnt)


def _layer(h, wl, wr, bl, act, src2, dst2, zrows, zcnt):
    y, z = _mm(h, wl, wr, bl)
    agg2, cntA, cntB = _sc_agg(y, src2, dst2, zrows, zcnt)
    return _combine(agg2, cntA.reshape(NP, 1), cntB.reshape(NP, 1), z, act)


def kernel(x, edge_index, W1l, W1r, W2l, W2r, W3l, W3r, b1l, b2l, b3l):
    src2 = edge_index[0].reshape(NW, NCHUNK, CH)
    dst2 = edge_index[1].reshape(NW, NCHUNK, CH)
    zrows = jnp.zeros((ROWS_PT, D), jnp.float32)
    zcnt = jnp.zeros((ROWS_PT,), jnp.float32)

    h = _layer(x, W1l, W1r, b1l, "relu", src2, dst2, zrows, zcnt)
    h = _layer(h, W2l, W2r, b2l, "relu", src2, dst2, zrows, zcnt)
    return _layer(h, W3l, W3r, b3l, "logsoftmax", src2, dst2, zrows, zcnt)
